# baseline (speedup 1.0000x reference)
import functools

import jax
import jax.numpy as jnp
from jax import lax
from jax.experimental import pallas as pl
from jax.experimental.pallas import tpu as pltpu
from jax.experimental.pallas import tpu_sc as plsc


def _gather_kernel(B, b_per_w, chunk, K, n_rounds, NC):
    mesh = plsc.VectorSubcoreMesh(core_axis_name="c", subcore_axis_name="s")
    group = K * chunk
    hc = chunk // 2

    @functools.partial(
        pl.kernel,
        mesh=mesh,
        out_type=jax.ShapeDtypeStruct((B // 2, 128), jnp.float32),
        compiler_params=pltpu.CompilerParams(use_tc_tiling_on_sc=False),
        scratch_types=[
            pltpu.VMEM((b_per_w,), jnp.int32),
            pltpu.VMEM((b_per_w,), jnp.int32),
            pltpu.VMEM((K, chunk, 128), jnp.float32),
            pltpu.VMEM((K, chunk, 128), jnp.float32),
            pltpu.SemaphoreType.DMA,
            pltpu.SemaphoreType.DMA,
            pltpu.SemaphoreType.DMA,
            pltpu.SemaphoreType.DMA,
        ],
    )
    def k(tab2_hbm, hidx_hbm, off_hbm, out_hbm, hidx_v, off_v, buf_a, buf_b,
          gsa, gsb, ssa, ssb):
        wid = lax.axis_index("s") * NC + lax.axis_index("c")
        base = wid * b_per_w
        pltpu.sync_copy(hidx_hbm.at[pl.ds(base, b_per_w)], hidx_v)
        pltpu.sync_copy(off_hbm.at[pl.ds(base, b_per_w)], off_v)

        def row0(o):
            return o * (2 * group)

        def issue_gathers(buf, sem, start):
            for t in range(K):
                pltpu.async_copy(
                    tab2_hbm.at[hidx_v.at[pl.ds(start + t * chunk, chunk)]],
                    buf.at[t],
                    sem,
                )

        def wait_gathers(buf, sem, start):
            for t in range(K):
                pltpu.make_async_copy(
                    tab2_hbm.at[hidx_v.at[pl.ds(start + t * chunk, chunk)]],
                    buf.at[t],
                    sem,
                ).wait()

        def select(buf, start):
            for t in range(K):
                blk = buf.at[t]

                def body(g, carry):
                    ov = off_v[pl.ds(start + t * chunk + g * 16, 16)]
                    for ti in range(16):
                        src_row = g * 16 + ti
                        dst_row = g * 8 + ti // 2
                        dlane = (ti % 2) * 64
                        off = ov[ti]
                        for j in range(4):
                            blk[dst_row, pl.ds(dlane + 16 * j, 16)] = blk[
                                src_row, pl.ds(off + 16 * j, 16)
                            ]
                    return carry

                lax.fori_loop(0, chunk // 16, body, 0)

        def issue_stores(buf, sem, start):
            for t in range(K):
                pltpu.async_copy(
                    buf.at[t, pl.ds(0, hc), :],
                    out_hbm.at[pl.ds((base + start + t * chunk) // 2, hc)],
                    sem,
                )

        def wait_stores(buf, sem, start):
            for t in range(K):
                pltpu.make_async_copy(
                    buf.at[t, pl.ds(0, hc), :],
                    out_hbm.at[pl.ds((base + start + t * chunk) // 2, hc)],
                    sem,
                ).wait()

        issue_gathers(buf_a, gsa, row0(0))

        def body(o, carry):
            a0 = row0(o)
            b0 = a0 + group
            issue_gathers(buf_b, gsb, b0)
            wait_gathers(buf_a, gsa, a0)
            select(buf_a, a0)
            issue_stores(buf_a, ssa, a0)
            wait_stores(buf_a, ssa, a0)
            issue_gathers(buf_a, gsa, row0(o + 1))
            wait_gathers(buf_b, gsb, b0)
            select(buf_b, b0)
            issue_stores(buf_b, ssb, b0)
            wait_stores(buf_b, ssb, b0)
            return carry

        lax.fori_loop(0, n_rounds - 1, body, 0)

        a0 = row0(n_rounds - 1)
        b0 = a0 + group
        issue_gathers(buf_b, gsb, b0)
        wait_gathers(buf_a, gsa, a0)
        select(buf_a, a0)
        issue_stores(buf_a, ssa, a0)
        wait_stores(buf_a, ssa, a0)
        wait_gathers(buf_b, gsb, b0)
        select(buf_b, b0)
        issue_stores(buf_b, ssb, b0)
        wait_stores(buf_b, ssb, b0)

    return k


def kernel(x, table):
    B0, S = x.shape
    V, D = table.shape
    B = B0 * S
    idx = x.reshape(B).astype(jnp.int32)
    hidx = idx >> 1
    off = (idx & 1) * D
    tab2 = table.reshape(V // 2, 2 * D)

    info = plsc.get_sparse_core_info()
    NC, NS = info.num_cores, info.num_subcores
    NW = NC * NS
    b_per_w = B // NW
    chunk = 128
    K = 2
    n_rounds = b_per_w // (2 * K * chunk)

    out2 = _gather_kernel(B, b_per_w, chunk, K, n_rounds, NC)(tab2, hidx, off)
    return out2.reshape(B0, S, D)


# speedup vs baseline: 1.2457x; 1.2457x over previous
import functools

import jax
import jax.numpy as jnp
from jax import lax
from jax.experimental import pallas as pl
from jax.experimental.pallas import tpu as pltpu
from jax.experimental.pallas import tpu_sc as plsc


def _gather_kernel(B, D, b_per_w, chunk, K, n_rounds, NC):
    mesh = plsc.VectorSubcoreMesh(core_axis_name="c", subcore_axis_name="s")
    group = K * chunk

    @functools.partial(
        pl.kernel,
        mesh=mesh,
        out_type=jax.ShapeDtypeStruct((B, D), jnp.float32),
        compiler_params=pltpu.CompilerParams(use_tc_tiling_on_sc=False),
        scratch_types=[
            pltpu.VMEM((b_per_w,), jnp.int32),
            pltpu.VMEM((K, chunk, D), jnp.float32),
            pltpu.VMEM((K, chunk, D), jnp.float32),
            pltpu.SemaphoreType.DMA,
            pltpu.SemaphoreType.DMA,
            pltpu.SemaphoreType.DMA,
            pltpu.SemaphoreType.DMA,
        ],
    )
    def k(tab_hbm, idx_hbm, out_hbm, idx_v, buf_a, buf_b, gsa, gsb, ssa, ssb):
        wid = lax.axis_index("s") * NC + lax.axis_index("c")
        base = wid * b_per_w
        pltpu.sync_copy(idx_hbm.at[pl.ds(base, b_per_w)], idx_v)

        def row0(o):
            return o * (2 * group)

        def issue_gathers(buf, sem, start):
            for t in range(K):
                pltpu.async_copy(
                    tab_hbm.at[idx_v.at[pl.ds(start + t * chunk, chunk)]],
                    buf.at[t],
                    sem,
                )

        def wait_gathers(buf, sem, start):
            for t in range(K):
                pltpu.make_async_copy(
                    tab_hbm.at[idx_v.at[pl.ds(start + t * chunk, chunk)]],
                    buf.at[t],
                    sem,
                ).wait()

        def issue_stores(buf, sem, start):
            for t in range(K):
                pltpu.async_copy(
                    buf.at[t],
                    out_hbm.at[pl.ds(base + start + t * chunk, chunk)],
                    sem,
                )

        def wait_stores(buf, sem, start):
            for t in range(K):
                pltpu.make_async_copy(
                    buf.at[t],
                    out_hbm.at[pl.ds(base + start + t * chunk, chunk)],
                    sem,
                ).wait()

        issue_gathers(buf_a, gsa, row0(0))

        def body(o, carry):
            a0 = row0(o)
            b0 = a0 + group
            issue_gathers(buf_b, gsb, b0)
            wait_gathers(buf_a, gsa, a0)
            issue_stores(buf_a, ssa, a0)
            wait_stores(buf_a, ssa, a0)
            issue_gathers(buf_a, gsa, row0(o + 1))
            wait_gathers(buf_b, gsb, b0)
            issue_stores(buf_b, ssb, b0)
            wait_stores(buf_b, ssb, b0)
            return carry

        lax.fori_loop(0, n_rounds - 1, body, 0)

        a0 = row0(n_rounds - 1)
        b0 = a0 + group
        issue_gathers(buf_b, gsb, b0)
        wait_gathers(buf_a, gsa, a0)
        issue_stores(buf_a, ssa, a0)
        wait_stores(buf_a, ssa, a0)
        wait_gathers(buf_b, gsb, b0)
        issue_stores(buf_b, ssb, b0)
        wait_stores(buf_b, ssb, b0)

    return k


def kernel(x, table):
    B0, S = x.shape
    V, D = table.shape
    B = B0 * S
    idx = x.reshape(B).astype(jnp.int32)

    info = plsc.get_sparse_core_info()
    NC, NS = info.num_cores, info.num_subcores
    NW = NC * NS
    b_per_w = B // NW
    chunk = 128
    K = 2
    n_rounds = b_per_w // (2 * K * chunk)

    out = _gather_kernel(B, D, b_per_w, chunk, K, n_rounds, NC)(table, idx)
    return out.reshape(B0, S, D)


# speedup vs baseline: 1.2457x; 1.0000x over previous
import functools

import jax
import jax.numpy as jnp
from jax import lax
from jax.experimental import pallas as pl
from jax.experimental.pallas import tpu as pltpu
from jax.experimental.pallas import tpu_sc as plsc


def _gather_kernel(B, D, b_per_w, chunk, K, n_rounds, NC):
    mesh = plsc.VectorSubcoreMesh(core_axis_name="c", subcore_axis_name="s")
    group = K * chunk

    @functools.partial(
        pl.kernel,
        mesh=mesh,
        out_type=jax.ShapeDtypeStruct((B, D), jnp.float32),
        compiler_params=pltpu.CompilerParams(use_tc_tiling_on_sc=False),
        scratch_types=[
            pltpu.VMEM((b_per_w,), jnp.int32),
            pltpu.VMEM((K, chunk, D), jnp.float32),
            pltpu.VMEM((K, chunk, D), jnp.float32),
            pltpu.SemaphoreType.DMA,
            pltpu.SemaphoreType.DMA,
            pltpu.SemaphoreType.DMA,
            pltpu.SemaphoreType.DMA,
        ],
    )
    def k(tab_hbm, idx_hbm, out_hbm, idx_v, buf_a, buf_b, gsa, gsb, ssa, ssb):
        wid = lax.axis_index("s") * NC + lax.axis_index("c")
        base = wid * b_per_w
        pltpu.sync_copy(idx_hbm.at[pl.ds(base, b_per_w)], idx_v)

        def row0(o):
            return o * (2 * group)

        def issue_gathers(buf, sem, start):
            for t in range(K):
                pltpu.async_copy(
                    tab_hbm.at[idx_v.at[pl.ds(start + t * chunk, chunk)]],
                    buf.at[t],
                    sem,
                )

        def wait_gathers(buf, sem, start):
            for t in range(K):
                pltpu.make_async_copy(
                    tab_hbm.at[idx_v.at[pl.ds(start + t * chunk, chunk)]],
                    buf.at[t],
                    sem,
                ).wait()

        def issue_stores(buf, sem, start):
            for t in range(K):
                pltpu.async_copy(
                    buf.at[t],
                    out_hbm.at[pl.ds(base + start + t * chunk, chunk)],
                    sem,
                )

        def wait_stores(buf, sem, start):
            for t in range(K):
                pltpu.make_async_copy(
                    buf.at[t],
                    out_hbm.at[pl.ds(base + start + t * chunk, chunk)],
                    sem,
                ).wait()

        issue_gathers(buf_a, gsa, row0(0))

        def body(o, carry):
            a0 = row0(o)
            b0 = a0 + group
            issue_gathers(buf_b, gsb, b0)
            wait_gathers(buf_a, gsa, a0)
            issue_stores(buf_a, ssa, a0)
            wait_stores(buf_a, ssa, a0)
            issue_gathers(buf_a, gsa, row0(o + 1))
            wait_gathers(buf_b, gsb, b0)
            issue_stores(buf_b, ssb, b0)
            wait_stores(buf_b, ssb, b0)
            return carry

        lax.fori_loop(0, n_rounds - 1, body, 0)

        a0 = row0(n_rounds - 1)
        b0 = a0 + group
        issue_gathers(buf_b, gsb, b0)
        wait_gathers(buf_a, gsa, a0)
        issue_stores(buf_a, ssa, a0)
        wait_stores(buf_a, ssa, a0)
        wait_gathers(buf_b, gsb, b0)
        issue_stores(buf_b, ssb, b0)
        wait_stores(buf_b, ssb, b0)

    return k


def kernel(x, table):
    B0, S = x.shape
    V, D = table.shape
    B = B0 * S
    idx = x.reshape(B).astype(jnp.int32)

    info = plsc.get_sparse_core_info()
    NC, NS = info.num_cores, info.num_subcores
    NW = NC * NS
    b_per_w = B // NW
    chunk = 128
    K = 4
    n_rounds = b_per_w // (2 * K * chunk)

    out = _gather_kernel(B, D, b_per_w, chunk, K, n_rounds, NC)(table, idx)
    return out.reshape(B0, S, D)
